# baseline (speedup 1.0000x reference)
import functools

import jax
import jax.numpy as jnp
from jax import lax
from jax.experimental import pallas as pl
from jax.experimental.pallas import tpu as pltpu
from jax.experimental.pallas import tpu_sc as plsc

D = 256
NE = 8192
NZ = 8192
NH = NZ // 2
BETA = 0.25

ROWS = 1024
CHUNK = 2048
LANES = 128
NSTEPS = NH // ROWS


def _bsq_body(emb_ref, b_ref):
    e = emb_ref[...]
    e2 = e * e
    s2 = e2[:, :LANES] + e2[:, LANES:]
    b_ref[...] = jnp.sum(s2.T, axis=0, keepdims=True)


_bsq_call = pl.pallas_call(
    _bsq_body,
    out_shape=jax.ShapeDtypeStruct((1, NE), jnp.float32),
)


def _make_argmin_body(final):
    def _argmin_body(zf_ref, emb_ref, b_ref, lin_ref, idx_ref, loss_ref):
        i = pl.program_id(0)

        zb = zf_ref[...]
        a = jnp.sum(zb * zb, axis=1, keepdims=True)
        a_bc = jnp.broadcast_to(a, (ROWS, LANES))
        zb2 = (zb + zb).astype(jnp.bfloat16)

        best_v = jnp.full((ROWS, LANES), jnp.inf, dtype=jnp.float32)
        best_t = jnp.zeros((ROWS, LANES), dtype=jnp.int32)

        for k in range(NE // CHUNK):
            ec = emb_ref[pl.ds(k * CHUNK, CHUNK), :].astype(jnp.bfloat16)
            c2 = lax.dot_general(zb2, ec, (((1,), (1,)), ((), ())),
                                 preferred_element_type=jnp.float32)
            bk = b_ref[:, pl.ds(k * CHUNK, CHUNK)]
            for t in range(CHUNK // LANES):
                gt = k * (CHUNK // LANES) + t
                ct = lax.slice(c2, (0, t * LANES), (ROWS, (t + 1) * LANES))
                bt = lax.slice(bk, (0, t * LANES), (1, (t + 1) * LANES))
                tv = a_bc + bt
                v = tv - ct
                upd = v < best_v
                best_t = jnp.where(upd, jnp.int32(gt), best_t)
                best_v = jnp.minimum(best_v, v)

        lane = lax.broadcasted_iota(jnp.int32, (ROWS, LANES), 1)
        best_j = best_t * LANES + lane
        m = jnp.min(best_v, axis=1, keepdims=True)
        im = jnp.min(jnp.where(best_v == m, best_j, jnp.int32(2 ** 30)),
                     axis=1, keepdims=True)
        idx_ref[...] = im.reshape(ROWS // LANES, LANES)

        part = jnp.sum(m)
        tot = jnp.where(i == 0, lin_ref[0, 0] + part, loss_ref[0, 0] + part)
        if final:
            scale = jnp.float32((1.0 + BETA) / (NZ * D))
            tot = jnp.where(i == NSTEPS - 1, tot * scale, tot)
        loss_ref[0, 0] = tot

    return _argmin_body


def _make_argmin_call(final):
    return pl.pallas_call(
        _make_argmin_body(final),
        grid=(NSTEPS,),
        in_specs=[
            pl.BlockSpec((ROWS, D), lambda i: (i, 0)),
            pl.BlockSpec((NE, D), lambda i: (0, 0)),
            pl.BlockSpec((1, NE), lambda i: (0, 0)),
            pl.BlockSpec(memory_space=pltpu.SMEM),
        ],
        out_specs=[
            pl.BlockSpec((ROWS // LANES, LANES), lambda i: (i, 0)),
            pl.BlockSpec(memory_space=pltpu.SMEM),
        ],
        out_shape=[
            jax.ShapeDtypeStruct((NH // LANES, LANES), jnp.int32),
            jax.ShapeDtypeStruct((1, 1), jnp.float32),
        ],
        compiler_params=pltpu.CompilerParams(
            dimension_semantics=("arbitrary",)),
    )


_argmin_half0 = _make_argmin_call(False)
_argmin_half1 = _make_argmin_call(True)


_info = plsc.get_sparse_core_info()
_NC, _NS = _info.num_cores, _info.num_subcores
_NW = _NC * _NS
_RPW = NH // _NW
_GC = 128


@functools.partial(
    pl.kernel,
    mesh=plsc.VectorSubcoreMesh(core_axis_name="c", subcore_axis_name="s"),
    out_type=jax.ShapeDtypeStruct((NH, D), jnp.float32),
    scratch_types=[
        pltpu.VMEM((_RPW // _GC, _GC), jnp.int32),
        pltpu.VMEM((_RPW // _GC, _GC, D), jnp.float32),
        pltpu.SemaphoreType.DMA,
    ],
)
def _gather_k(emb_hbm, idx_hbm, out_hbm, idx_v, rows_v, sem):
    wid = lax.axis_index("s") * _NC + lax.axis_index("c")
    base = wid * _RPW
    nch = _RPW // _GC
    for j in range(nch):
        pltpu.sync_copy(idx_hbm.at[pl.ds(base + j * _GC, _GC)], idx_v.at[j])
    cps = [pltpu.async_copy(emb_hbm.at[idx_v.at[j]], rows_v.at[j], sem)
           for j in range(nch)]
    for j in range(nch):
        cps[j].wait()
        pltpu.sync_copy(rows_v.at[j], out_hbm.at[pl.ds(base + j * _GC, _GC)])


def _st_body(zf_ref, zq_ref, out_ref):
    zb = zf_ref[...]
    out_ref[...] = zb + (zq_ref[...] - zb)


_ST_G = 2

_st_call = pl.pallas_call(
    _st_body,
    grid=(_ST_G,),
    in_specs=[
        pl.BlockSpec((NH // _ST_G, D), lambda i: (i, 0)),
        pl.BlockSpec((NH // _ST_G, D), lambda i: (i, 0)),
    ],
    out_specs=pl.BlockSpec((NH // _ST_G, D), lambda i: (i, 0)),
    out_shape=jax.ShapeDtypeStruct((NH, D), jnp.float32),
)


def kernel(z, emb):
    B, T, H, W, d = z.shape
    zf = z.reshape(-1, d)
    zf0, zf1 = zf[:NH], zf[NH:]
    b = _bsq_call(emb)
    zero = jnp.zeros((1, 1), jnp.float32)
    idx0, l0 = _argmin_half0(zf0, emb, b, zero)
    i0 = idx0.reshape(-1)
    z_q0 = _gather_k(emb, i0)
    idx1, loss = _argmin_half1(zf1, emb, b, l0)
    i1 = idx1.reshape(-1)
    z_q1 = _gather_k(emb, i1)
    st0 = _st_call(zf0, z_q0)
    st1 = _st_call(zf1, z_q1)
    z_q_st = jnp.concatenate([st0, st1], axis=0)
    idx = jnp.concatenate([i0, i1], axis=0)
    return (z_q_st.reshape(B, T, H, W, d), loss.reshape(()),
            idx.reshape(B, T, H, W))


# speedup vs baseline: 1.2102x; 1.2102x over previous
import functools

import jax
import jax.numpy as jnp
from jax import lax
from jax.experimental import pallas as pl
from jax.experimental.pallas import tpu as pltpu
from jax.experimental.pallas import tpu_sc as plsc

D = 256
NE = 8192
NZ = 8192
BETA = 0.25

ROWS = 1024
CHUNK = 2048
LANES = 128
NSTEPS = NZ // ROWS


def _bsq_body(emb_ref, b_ref):
    e = emb_ref[...]
    e2 = e * e
    s2 = e2[:, :LANES] + e2[:, LANES:]
    b_ref[...] = jnp.sum(s2.T, axis=0, keepdims=True)


_bsq_call = pl.pallas_call(
    _bsq_body,
    out_shape=jax.ShapeDtypeStruct((1, NE), jnp.float32),
)


def _argmin_body(zf_ref, emb_ref, b_ref, idx_ref, loss_ref):
    i = pl.program_id(0)

    zb = zf_ref[...]
    a = jnp.sum(zb * zb, axis=1, keepdims=True)
    a_bc = jnp.broadcast_to(a, (ROWS, LANES))
    zb2 = (zb + zb).astype(jnp.bfloat16)

    best_v = jnp.full((ROWS, LANES), jnp.inf, dtype=jnp.float32)
    best_t = jnp.zeros((ROWS, LANES), dtype=jnp.int32)

    for k in range(NE // CHUNK):
        ec = emb_ref[pl.ds(k * CHUNK, CHUNK), :].astype(jnp.bfloat16)
        c2 = lax.dot_general(zb2, ec, (((1,), (1,)), ((), ())),
                             preferred_element_type=jnp.float32)
        bk = b_ref[:, pl.ds(k * CHUNK, CHUNK)]
        for t in range(CHUNK // LANES):
            gt = k * (CHUNK // LANES) + t
            ct = lax.slice(c2, (0, t * LANES), (ROWS, (t + 1) * LANES))
            bt = lax.slice(bk, (0, t * LANES), (1, (t + 1) * LANES))
            tv = a_bc + bt
            v = tv - ct
            upd = v < best_v
            best_t = jnp.where(upd, jnp.int32(gt), best_t)
            best_v = jnp.minimum(best_v, v)

    lane = lax.broadcasted_iota(jnp.int32, (ROWS, LANES), 1)
    best_j = best_t * LANES + lane
    m = jnp.min(best_v, axis=1, keepdims=True)
    im = jnp.min(jnp.where(best_v == m, best_j, jnp.int32(2 ** 30)),
                 axis=1, keepdims=True)
    idx_ref[...] = im.reshape(ROWS // LANES, LANES)

    part = jnp.sum(m)
    tot = jnp.where(i == 0, part, loss_ref[0, 0] + part)
    scale = jnp.float32((1.0 + BETA) / (NZ * D))
    loss_ref[0, 0] = jnp.where(i == NSTEPS - 1, tot * scale, tot)


_argmin_call = pl.pallas_call(
    _argmin_body,
    grid=(NSTEPS,),
    in_specs=[
        pl.BlockSpec((ROWS, D), lambda i: (i, 0)),
        pl.BlockSpec((NE, D), lambda i: (0, 0)),
        pl.BlockSpec((1, NE), lambda i: (0, 0)),
    ],
    out_specs=[
        pl.BlockSpec((ROWS // LANES, LANES), lambda i: (i, 0)),
        pl.BlockSpec(memory_space=pltpu.SMEM),
    ],
    out_shape=[
        jax.ShapeDtypeStruct((NZ // LANES, LANES), jnp.int32),
        jax.ShapeDtypeStruct((1, 1), jnp.float32),
    ],
    compiler_params=pltpu.CompilerParams(
        dimension_semantics=("arbitrary",)),
)


_info = plsc.get_sparse_core_info()
_NC, _NS = _info.num_cores, _info.num_subcores
_NW = _NC * _NS
_RPW = NZ // _NW
_GC = 128


@functools.partial(
    pl.kernel,
    mesh=plsc.VectorSubcoreMesh(core_axis_name="c", subcore_axis_name="s"),
    out_type=jax.ShapeDtypeStruct((NZ, D), jnp.float32),
    scratch_types=[
        pltpu.VMEM((_RPW // _GC, _GC), jnp.int32),
        pltpu.VMEM((_RPW // _GC, _GC, D), jnp.float32),
        pltpu.SemaphoreType.DMA,
    ],
)
def _gather_k(emb_hbm, idx_hbm, out_hbm, idx_v, rows_v, sem):
    wid = lax.axis_index("s") * _NC + lax.axis_index("c")
    base = wid * _RPW
    nch = _RPW // _GC
    for j in range(nch):
        pltpu.sync_copy(idx_hbm.at[pl.ds(base + j * _GC, _GC)], idx_v.at[j])
    cps = [pltpu.async_copy(emb_hbm.at[idx_v.at[j]], rows_v.at[j], sem)
           for j in range(nch)]
    for j in range(nch):
        cps[j].wait()
        pltpu.sync_copy(rows_v.at[j], out_hbm.at[pl.ds(base + j * _GC, _GC)])


def _st_body(zf_ref, zq_ref, out_ref):
    zb = zf_ref[...]
    out_ref[...] = zb + (zq_ref[...] - zb)


_ST_G = 4

_st_call = pl.pallas_call(
    _st_body,
    grid=(_ST_G,),
    in_specs=[
        pl.BlockSpec((NZ // _ST_G, D), lambda i: (i, 0)),
        pl.BlockSpec((NZ // _ST_G, D), lambda i: (i, 0)),
    ],
    out_specs=pl.BlockSpec((NZ // _ST_G, D), lambda i: (i, 0)),
    out_shape=jax.ShapeDtypeStruct((NZ, D), jnp.float32),
)


def kernel(z, emb):
    B, T, H, W, d = z.shape
    zf = z.reshape(-1, d)
    b = _bsq_call(emb)
    idx2, loss = _argmin_call(zf, emb, b)
    idx = idx2.reshape(-1)
    z_q = _gather_k(emb, idx)
    z_q_st = _st_call(zf, z_q)
    return (z_q_st.reshape(B, T, H, W, d), loss.reshape(()),
            idx.reshape(B, T, H, W))


# speedup vs baseline: 1.2693x; 1.0489x over previous
import functools

import jax
import jax.numpy as jnp
from jax import lax
from jax.experimental import pallas as pl
from jax.experimental.pallas import tpu as pltpu
from jax.experimental.pallas import tpu_sc as plsc

D = 256
NE = 8192
NZ = 8192
BETA = 0.25

ROWS = 2048
CHUNK = 2048
LANES = 128
NSTEPS = NZ // ROWS


def _bsq_body(emb_ref, b_ref):
    e = emb_ref[...]
    e2 = e * e
    s2 = e2[:, :LANES] + e2[:, LANES:]
    b_ref[...] = jnp.sum(s2.T, axis=0, keepdims=True)


_bsq_call = pl.pallas_call(
    _bsq_body,
    out_shape=jax.ShapeDtypeStruct((1, NE), jnp.float32),
)


def _argmin_body(zf_ref, emb_ref, b_ref, idx_ref, loss_ref):
    i = pl.program_id(0)

    zb = zf_ref[...]
    a = jnp.sum(zb * zb, axis=1, keepdims=True)
    a_bc = jnp.broadcast_to(a, (ROWS, LANES))
    zb2 = (zb + zb).astype(jnp.bfloat16)

    best_v = jnp.full((ROWS, LANES), jnp.inf, dtype=jnp.float32)
    best_t = jnp.zeros((ROWS, LANES), dtype=jnp.int32)

    for k in range(NE // CHUNK):
        ec = emb_ref[pl.ds(k * CHUNK, CHUNK), :].astype(jnp.bfloat16)
        c2 = lax.dot_general(zb2, ec, (((1,), (1,)), ((), ())),
                             preferred_element_type=jnp.float32)
        bk = b_ref[:, pl.ds(k * CHUNK, CHUNK)]
        for t in range(CHUNK // LANES):
            gt = k * (CHUNK // LANES) + t
            ct = lax.slice(c2, (0, t * LANES), (ROWS, (t + 1) * LANES))
            bt = lax.slice(bk, (0, t * LANES), (1, (t + 1) * LANES))
            tv = a_bc + bt
            v = tv - ct
            upd = v < best_v
            best_t = jnp.where(upd, jnp.int32(gt), best_t)
            best_v = jnp.minimum(best_v, v)

    lane = lax.broadcasted_iota(jnp.int32, (ROWS, LANES), 1)
    best_j = best_t * LANES + lane
    m = jnp.min(best_v, axis=1, keepdims=True)
    im = jnp.min(jnp.where(best_v == m, best_j, jnp.int32(2 ** 30)),
                 axis=1, keepdims=True)
    idx_ref[0] = im.reshape(ROWS // LANES, LANES)

    part = jnp.sum(m)
    tot = jnp.where(i == 0, part, loss_ref[0, 0] + part)
    scale = jnp.float32((1.0 + BETA) / (NZ * D))
    loss_ref[0, 0] = jnp.where(i == NSTEPS - 1, tot * scale, tot)


_argmin_call = pl.pallas_call(
    _argmin_body,
    grid=(NSTEPS,),
    in_specs=[
        pl.BlockSpec((ROWS, D), lambda i: (i, 0)),
        pl.BlockSpec((NE, D), lambda i: (0, 0)),
        pl.BlockSpec((1, NE), lambda i: (0, 0)),
    ],
    out_specs=[
        pl.BlockSpec((1, ROWS // LANES, LANES), lambda i: (i, 0, 0)),
        pl.BlockSpec(memory_space=pltpu.SMEM),
    ],
    out_shape=[
        jax.ShapeDtypeStruct((NSTEPS, ROWS // LANES, LANES), jnp.int32),
        jax.ShapeDtypeStruct((1, 1), jnp.float32),
    ],
    compiler_params=pltpu.CompilerParams(
        dimension_semantics=("arbitrary",)),
)


_info = plsc.get_sparse_core_info()
_NC, _NS = _info.num_cores, _info.num_subcores
_NW = _NC * _NS
_RPW = NZ // _NW
_GC = 128


@functools.partial(
    pl.kernel,
    mesh=plsc.VectorSubcoreMesh(core_axis_name="c", subcore_axis_name="s"),
    out_type=jax.ShapeDtypeStruct((NZ, D), jnp.float32),
    scratch_types=[
        pltpu.VMEM((_RPW // _GC, _GC), jnp.int32),
        pltpu.VMEM((_RPW // _GC, _GC, D), jnp.float32),
        pltpu.SemaphoreType.DMA,
    ],
)
def _gather_k(emb_hbm, idx_hbm, out_hbm, idx_v, rows_v, sem):
    wid = lax.axis_index("s") * _NC + lax.axis_index("c")
    base = wid * _RPW
    nch = _RPW // _GC
    for j in range(nch):
        pltpu.sync_copy(idx_hbm.at[pl.ds(base + j * _GC, _GC)], idx_v.at[j])
    cps = [pltpu.async_copy(emb_hbm.at[idx_v.at[j]], rows_v.at[j], sem)
           for j in range(nch)]
    for j in range(nch):
        cps[j].wait()
        pltpu.sync_copy(rows_v.at[j], out_hbm.at[pl.ds(base + j * _GC, _GC)])


def _st_body(zf_ref, zq_ref, out_ref):
    zb = zf_ref[...]
    out_ref[...] = zb + (zq_ref[...] - zb)


_ST_G = 4

_st_call = pl.pallas_call(
    _st_body,
    grid=(_ST_G,),
    in_specs=[
        pl.BlockSpec((NZ // _ST_G, D), lambda i: (i, 0)),
        pl.BlockSpec((NZ // _ST_G, D), lambda i: (i, 0)),
    ],
    out_specs=pl.BlockSpec((NZ // _ST_G, D), lambda i: (i, 0)),
    out_shape=jax.ShapeDtypeStruct((NZ, D), jnp.float32),
)


def kernel(z, emb):
    B, T, H, W, d = z.shape
    zf = z.reshape(-1, d)
    b = _bsq_call(emb)
    idx2, loss = _argmin_call(zf, emb, b)
    idx = idx2.reshape(-1)
    z_q = _gather_k(emb, idx)
    z_q_st = _st_call(zf, z_q)
    return (z_q_st.reshape(B, T, H, W, d), loss.reshape(()),
            idx.reshape(B, T, H, W))
